# baseline (speedup 1.0000x reference)
import functools

import jax
import jax.numpy as jnp
from jax.experimental import pallas as pl
from jax.experimental.pallas import tpu as pltpu

D_MODEL = 4096
OUT_FEATURES = 4096
NUM_EXPERTS = 8
TOP_K = 2
LORA_RANK = 16
SCALING = 32 / 16

TN = 512


def _fused_kernel(x_ref, wb_ref, b_ref, wg_ref, aall_ref, ball_ref,
                  out_ref, hw_ref):
    n = pl.program_id(0)

    @pl.when(n == 0)
    def _router_and_lora_a():
        CH = 512

        def body(c, _):
            x = x_ref[pl.ds(c * CH, CH), :]
            logits = jax.lax.dot_general(
                x, wg_ref[...], (((1,), (1,)), ((), ())),
                preferred_element_type=jnp.float32)
            mx = jnp.max(logits, axis=1, keepdims=True)
            e = jnp.exp(logits - mx)
            probs = e / jnp.sum(e, axis=1, keepdims=True)
            iota8 = jax.lax.broadcasted_iota(jnp.int32, probs.shape, 1)
            m1 = jnp.max(probs, axis=1, keepdims=True)
            i1 = jnp.min(jnp.where(probs == m1, iota8, NUM_EXPERTS),
                         axis=1, keepdims=True)
            masked = jnp.where(iota8 == i1, -1.0, probs)
            m2 = jnp.max(masked, axis=1, keepdims=True)
            i2 = jnp.min(jnp.where(masked == m2, iota8, NUM_EXPERTS),
                         axis=1, keepdims=True)
            scale = SCALING / (m1 + m2)
            grp = jax.lax.broadcasted_iota(
                jnp.int32, (CH, NUM_EXPERTS * LORA_RANK), 1) // LORA_RANK
            w128 = (jnp.where(grp == i1, m1, 0.0)
                    + jnp.where(grp == i2, m2, 0.0)) * scale
            h = jax.lax.dot_general(
                x, aall_ref[...], (((1,), (1,)), ((), ())),
                preferred_element_type=jnp.float32)
            hw_ref[pl.ds(c * CH, CH), :] = (h * w128).astype(jnp.bfloat16)
            return _

        jax.lax.fori_loop(0, x_ref.shape[0] // CH, body, 0, unroll=False)

    CHM = 512

    def body_main(c, _):
        rows = pl.ds(c * CHM, CHM)
        acc = jax.lax.dot_general(
            x_ref[rows, :], wb_ref[...], (((1,), (1,)), ((), ())),
            preferred_element_type=jnp.float32)
        acc += jax.lax.dot_general(
            hw_ref[rows, :], ball_ref[...],
            (((1,), (0,)), ((), ())),
            preferred_element_type=jnp.float32)
        out_ref[rows, :] = acc + b_ref[...]
        return _

    jax.lax.fori_loop(0, x_ref.shape[0] // CHM, body_main, 0, unroll=False)


@jax.jit
def kernel(inputs, W_base, b_base, W_gate, A, B):
    lead = inputs.shape[:-1]
    x = inputs.reshape(-1, D_MODEL).astype(jnp.bfloat16)
    M = x.shape[0]
    A_all = A.reshape(NUM_EXPERTS * LORA_RANK, D_MODEL).astype(jnp.bfloat16)
    B_all = B.transpose(0, 2, 1).reshape(
        NUM_EXPERTS * LORA_RANK, OUT_FEATURES).astype(jnp.bfloat16)
    W_base = W_base.astype(jnp.bfloat16)
    W_gate = W_gate.astype(jnp.bfloat16)
    b2 = b_base.reshape(1, OUT_FEATURES)

    grid = (OUT_FEATURES // TN,)
    out = pl.pallas_call(
        _fused_kernel,
        grid=grid,
        in_specs=[
            pl.BlockSpec((M, D_MODEL), lambda n: (0, 0)),
            pl.BlockSpec((TN, D_MODEL), lambda n: (n, 0)),
            pl.BlockSpec((1, TN), lambda n: (0, n)),
            pl.BlockSpec((NUM_EXPERTS, D_MODEL), lambda n: (0, 0)),
            pl.BlockSpec((NUM_EXPERTS * LORA_RANK, D_MODEL),
                         lambda n: (0, 0)),
            pl.BlockSpec((NUM_EXPERTS * LORA_RANK, TN),
                         lambda n: (0, n)),
        ],
        out_specs=pl.BlockSpec((M, TN), lambda n: (0, n)),
        out_shape=jax.ShapeDtypeStruct((M, OUT_FEATURES), jnp.float32),
        scratch_shapes=[pltpu.VMEM((M, NUM_EXPERTS * LORA_RANK),
                                   jnp.bfloat16)],
        compiler_params=pltpu.CompilerParams(
            dimension_semantics=("arbitrary",),
            vmem_limit_bytes=63 * 1024 * 1024,
        ),
    )(x, W_base, b2, W_gate, A_all, B_all)
    return out.reshape(lead + (OUT_FEATURES,))


# speedup vs baseline: 1.0445x; 1.0445x over previous
import jax
import jax.numpy as jnp
from jax.experimental import pallas as pl
from jax.experimental.pallas import tpu as pltpu

D_MODEL = 4096
OUT_FEATURES = 4096
NUM_EXPERTS = 8
TOP_K = 2
LORA_RANK = 16
SCALING = 32 / 16

TN = 512
CH = 512


def _router_chunk(x, wg_ref, aall_ref):
    logits = jax.lax.dot_general(
        x, wg_ref[...], (((1,), (1,)), ((), ())),
        preferred_element_type=jnp.float32)
    mx = jnp.max(logits, axis=1, keepdims=True)
    e = jnp.exp(logits - mx)
    probs = e / jnp.sum(e, axis=1, keepdims=True)
    iota8 = jax.lax.broadcasted_iota(jnp.int32, probs.shape, 1)
    m1 = jnp.max(probs, axis=1, keepdims=True)
    i1 = jnp.min(jnp.where(probs == m1, iota8, NUM_EXPERTS),
                 axis=1, keepdims=True)
    masked = jnp.where(iota8 == i1, -1.0, probs)
    m2 = jnp.max(masked, axis=1, keepdims=True)
    i2 = jnp.min(jnp.where(masked == m2, iota8, NUM_EXPERTS),
                 axis=1, keepdims=True)
    scale = SCALING / (m1 + m2)
    grp = jax.lax.broadcasted_iota(
        jnp.int32, (CH, NUM_EXPERTS * LORA_RANK), 1) // LORA_RANK
    w128 = (jnp.where(grp == i1, m1, 0.0)
            + jnp.where(grp == i2, m2, 0.0)) * scale
    h = jax.lax.dot_general(
        x, aall_ref[...], (((1,), (1,)), ((), ())),
        preferred_element_type=jnp.float32)
    return (h * w128).astype(jnp.bfloat16)


def _fused_kernel(x_ref, wb_ref, b_ref, wg_ref, aall_ref, ball_ref,
                  out_ref, hw_ref):
    n = pl.program_id(0)
    nchunks = x_ref.shape[0] // CH

    def main_chunk(c):
        rows = pl.ds(c * CH, CH)
        acc = jax.lax.dot_general(
            x_ref[rows, :], wb_ref[...], (((1,), (1,)), ((), ())),
            preferred_element_type=jnp.float32)
        acc += jax.lax.dot_general(
            hw_ref[rows, :], ball_ref[...], (((1,), (0,)), ((), ())),
            preferred_element_type=jnp.float32)
        out_ref[rows, :] = acc + b_ref[...]

    @pl.when(n == 0)
    def _first():
        def body(c, carry):
            rows = pl.ds(c * CH, CH)
            hw_ref[rows, :] = _router_chunk(x_ref[rows, :],
                                            wg_ref, aall_ref)
            main_chunk(c)
            return carry

        jax.lax.fori_loop(0, nchunks, body, 0, unroll=True)

    @pl.when(n != 0)
    def _rest():
        def body(c, carry):
            main_chunk(c)
            return carry

        jax.lax.fori_loop(0, nchunks, body, 0, unroll=True)


@jax.jit
def kernel(inputs, W_base, b_base, W_gate, A, B):
    lead = inputs.shape[:-1]
    x = inputs.reshape(-1, D_MODEL).astype(jnp.bfloat16)
    M = x.shape[0]
    A_all = A.reshape(NUM_EXPERTS * LORA_RANK, D_MODEL).astype(jnp.bfloat16)
    B_all = B.transpose(0, 2, 1).reshape(
        NUM_EXPERTS * LORA_RANK, OUT_FEATURES).astype(jnp.bfloat16)
    W_base = W_base.astype(jnp.bfloat16)
    W_gate = W_gate.astype(jnp.bfloat16)
    b2 = b_base.reshape(1, OUT_FEATURES)

    grid = (OUT_FEATURES // TN,)
    out = pl.pallas_call(
        _fused_kernel,
        grid=grid,
        in_specs=[
            pl.BlockSpec((M, D_MODEL), lambda n: (0, 0)),
            pl.BlockSpec((TN, D_MODEL), lambda n: (n, 0)),
            pl.BlockSpec((1, TN), lambda n: (0, n)),
            pl.BlockSpec((NUM_EXPERTS, D_MODEL), lambda n: (0, 0)),
            pl.BlockSpec((NUM_EXPERTS * LORA_RANK, D_MODEL),
                         lambda n: (0, 0)),
            pl.BlockSpec((NUM_EXPERTS * LORA_RANK, TN),
                         lambda n: (0, n)),
        ],
        out_specs=pl.BlockSpec((M, TN), lambda n: (0, n)),
        out_shape=jax.ShapeDtypeStruct((M, OUT_FEATURES), jnp.float32),
        scratch_shapes=[pltpu.VMEM((M, NUM_EXPERTS * LORA_RANK),
                                   jnp.bfloat16)],
        compiler_params=pltpu.CompilerParams(
            dimension_semantics=("arbitrary",),
            vmem_limit_bytes=63 * 1024 * 1024,
        ),
    )(x, W_base, b2, W_gate, A_all, B_all)
    return out.reshape(lead + (OUT_FEATURES,))


# speedup vs baseline: 1.1577x; 1.1084x over previous
import jax
import jax.numpy as jnp
from jax.experimental import pallas as pl
from jax.experimental.pallas import tpu as pltpu

D_MODEL = 4096
OUT_FEATURES = 4096
NUM_EXPERTS = 8
TOP_K = 2
LORA_RANK = 16
SCALING = 32 / 16

TN = 256
CH = 512


def _router_chunk(x, wg_ref, aall_ref):
    logits = jax.lax.dot_general(
        x, wg_ref[...], (((1,), (1,)), ((), ())),
        preferred_element_type=jnp.float32)
    mx = jnp.max(logits, axis=1, keepdims=True)
    e = jnp.exp(logits - mx)
    probs = e / jnp.sum(e, axis=1, keepdims=True)
    iota8 = jax.lax.broadcasted_iota(jnp.int32, probs.shape, 1)
    m1 = jnp.max(probs, axis=1, keepdims=True)
    i1 = jnp.min(jnp.where(probs == m1, iota8, NUM_EXPERTS),
                 axis=1, keepdims=True)
    masked = jnp.where(iota8 == i1, -1.0, probs)
    m2 = jnp.max(masked, axis=1, keepdims=True)
    i2 = jnp.min(jnp.where(masked == m2, iota8, NUM_EXPERTS),
                 axis=1, keepdims=True)
    scale = SCALING / (m1 + m2)
    grp = jax.lax.broadcasted_iota(
        jnp.int32, (CH, NUM_EXPERTS * LORA_RANK), 1) // LORA_RANK
    w128 = (jnp.where(grp == i1, m1, 0.0)
            + jnp.where(grp == i2, m2, 0.0)) * scale
    h = jax.lax.dot_general(
        x, aall_ref[...], (((1,), (1,)), ((), ())),
        preferred_element_type=jnp.float32)
    return (h * w128).astype(jnp.bfloat16)


def _fused_kernel(x_ref, wb_ref, b_ref, wg_ref, aall_ref, ball_ref,
                  out_ref, hw_ref, w_ref):
    n = pl.program_id(0)
    nchunks = x_ref.shape[0] // CH

    w_ref[...] = wb_ref[...].astype(jnp.bfloat16)

    def main_chunk(c):
        rows = pl.ds(c * CH, CH)
        acc = jax.lax.dot_general(
            x_ref[rows, :], w_ref[...], (((1,), (1,)), ((), ())),
            preferred_element_type=jnp.float32)
        acc += jax.lax.dot_general(
            hw_ref[rows, :], ball_ref[...], (((1,), (0,)), ((), ())),
            preferred_element_type=jnp.float32)
        out_ref[rows, :] = acc + b_ref[...]

    @pl.when(n == 0)
    def _first():
        def body(c, carry):
            rows = pl.ds(c * CH, CH)
            hw_ref[rows, :] = _router_chunk(x_ref[rows, :],
                                            wg_ref, aall_ref)
            main_chunk(c)
            return carry

        jax.lax.fori_loop(0, nchunks, body, 0, unroll=True)

    @pl.when(n != 0)
    def _rest():
        def body(c, carry):
            main_chunk(c)
            return carry

        jax.lax.fori_loop(0, nchunks, body, 0, unroll=True)


@jax.jit
def kernel(inputs, W_base, b_base, W_gate, A, B):
    lead = inputs.shape[:-1]
    x = inputs.reshape(-1, D_MODEL).astype(jnp.bfloat16)
    M = x.shape[0]
    A_all = A.reshape(NUM_EXPERTS * LORA_RANK, D_MODEL).astype(jnp.bfloat16)
    B_all = B.transpose(0, 2, 1).reshape(
        NUM_EXPERTS * LORA_RANK, OUT_FEATURES).astype(jnp.bfloat16)
    W_gate = W_gate.astype(jnp.bfloat16)
    b2 = b_base.reshape(1, OUT_FEATURES)

    grid = (OUT_FEATURES // TN,)
    out = pl.pallas_call(
        _fused_kernel,
        grid=grid,
        in_specs=[
            pl.BlockSpec((M, D_MODEL), lambda n: (0, 0)),
            pl.BlockSpec((TN, D_MODEL), lambda n: (n, 0)),
            pl.BlockSpec((1, TN), lambda n: (0, n)),
            pl.BlockSpec((NUM_EXPERTS, D_MODEL), lambda n: (0, 0)),
            pl.BlockSpec((NUM_EXPERTS * LORA_RANK, D_MODEL),
                         lambda n: (0, 0)),
            pl.BlockSpec((NUM_EXPERTS * LORA_RANK, TN),
                         lambda n: (0, n)),
        ],
        out_specs=pl.BlockSpec((M, TN), lambda n: (0, n)),
        out_shape=jax.ShapeDtypeStruct((M, OUT_FEATURES), jnp.float32),
        scratch_shapes=[pltpu.VMEM((M, NUM_EXPERTS * LORA_RANK),
                                   jnp.bfloat16),
                        pltpu.VMEM((TN, D_MODEL), jnp.bfloat16)],
        compiler_params=pltpu.CompilerParams(
            dimension_semantics=("arbitrary",),
            vmem_limit_bytes=63 * 1024 * 1024,
        ),
    )(x, W_base, b2, W_gate, A_all, B_all)
    return out.reshape(lead + (OUT_FEATURES,))


# speedup vs baseline: 1.3085x; 1.1303x over previous
import jax
import jax.numpy as jnp
from jax.experimental import pallas as pl
from jax.experimental.pallas import tpu as pltpu

D_MODEL = 4096
OUT_FEATURES = 4096
NUM_EXPERTS = 8
TOP_K = 2
LORA_RANK = 16
SCALING = 32 / 16

TN = 256
CH = 1024
RC = 1024


def _prep_kernel(x_ref, wg_ref, aall_ref, xbf_ref, hw_ref):
    x = x_ref[...].astype(jnp.bfloat16)
    xbf_ref[...] = x
    logits = jax.lax.dot_general(
        x, wg_ref[...], (((1,), (1,)), ((), ())),
        preferred_element_type=jnp.float32)
    mx = jnp.max(logits, axis=1, keepdims=True)
    e = jnp.exp(logits - mx)
    probs = e / jnp.sum(e, axis=1, keepdims=True)
    iota8 = jax.lax.broadcasted_iota(jnp.int32, probs.shape, 1)
    m1 = jnp.max(probs, axis=1, keepdims=True)
    i1 = jnp.min(jnp.where(probs == m1, iota8, NUM_EXPERTS),
                 axis=1, keepdims=True)
    masked = jnp.where(iota8 == i1, -1.0, probs)
    m2 = jnp.max(masked, axis=1, keepdims=True)
    i2 = jnp.min(jnp.where(masked == m2, iota8, NUM_EXPERTS),
                 axis=1, keepdims=True)
    scale = SCALING / (m1 + m2)
    grp = jax.lax.broadcasted_iota(
        jnp.int32, (RC, NUM_EXPERTS * LORA_RANK), 1) // LORA_RANK
    w128 = (jnp.where(grp == i1, m1, 0.0)
            + jnp.where(grp == i2, m2, 0.0)) * scale
    h = jax.lax.dot_general(
        x, aall_ref[...], (((1,), (1,)), ((), ())),
        preferred_element_type=jnp.float32)
    hw_ref[...] = (h * w128).astype(jnp.bfloat16)


def _mm_kernel(x_ref, wb_ref, b_ref, ball_ref, hw_ref, out_ref, w_ref):
    w_ref[...] = wb_ref[...].astype(jnp.bfloat16)
    nchunks = x_ref.shape[0] // CH

    def body(c, carry):
        rows = pl.ds(c * CH, CH)
        acc = jax.lax.dot_general(
            x_ref[rows, :], w_ref[...], (((1,), (1,)), ((), ())),
            preferred_element_type=jnp.float32)
        acc += jax.lax.dot_general(
            hw_ref[rows, :], ball_ref[...], (((1,), (0,)), ((), ())),
            preferred_element_type=jnp.float32)
        out_ref[rows, :] = acc + b_ref[...]
        return carry

    jax.lax.fori_loop(0, nchunks, body, 0, unroll=True)


@jax.jit
def kernel(inputs, W_base, b_base, W_gate, A, B):
    lead = inputs.shape[:-1]
    x = inputs.reshape(-1, D_MODEL)
    M = x.shape[0]
    A_all = A.reshape(NUM_EXPERTS * LORA_RANK, D_MODEL).astype(jnp.bfloat16)
    B_all = B.transpose(0, 2, 1).reshape(
        NUM_EXPERTS * LORA_RANK, OUT_FEATURES).astype(jnp.bfloat16)
    W_gate = W_gate.astype(jnp.bfloat16)
    b2 = b_base.reshape(1, OUT_FEATURES)

    xbf, hw = pl.pallas_call(
        _prep_kernel,
        grid=(M // RC,),
        in_specs=[
            pl.BlockSpec((RC, D_MODEL), lambda r: (r, 0)),
            pl.BlockSpec((NUM_EXPERTS, D_MODEL), lambda r: (0, 0)),
            pl.BlockSpec((NUM_EXPERTS * LORA_RANK, D_MODEL),
                         lambda r: (0, 0)),
        ],
        out_specs=[
            pl.BlockSpec((RC, D_MODEL), lambda r: (r, 0)),
            pl.BlockSpec((RC, NUM_EXPERTS * LORA_RANK), lambda r: (r, 0)),
        ],
        out_shape=[
            jax.ShapeDtypeStruct((M, D_MODEL), jnp.bfloat16),
            jax.ShapeDtypeStruct((M, NUM_EXPERTS * LORA_RANK),
                                 jnp.bfloat16),
        ],
        compiler_params=pltpu.CompilerParams(
            dimension_semantics=("arbitrary",),
        ),
    )(x, W_gate, A_all)

    out = pl.pallas_call(
        _mm_kernel,
        grid=(OUT_FEATURES // TN,),
        in_specs=[
            pl.BlockSpec((M, D_MODEL), lambda n: (0, 0)),
            pl.BlockSpec((TN, D_MODEL), lambda n: (n, 0)),
            pl.BlockSpec((1, TN), lambda n: (0, n)),
            pl.BlockSpec((NUM_EXPERTS * LORA_RANK, TN),
                         lambda n: (0, n)),
            pl.BlockSpec((M, NUM_EXPERTS * LORA_RANK),
                         lambda n: (0, 0)),
        ],
        out_specs=pl.BlockSpec((M, TN), lambda n: (0, n)),
        out_shape=jax.ShapeDtypeStruct((M, OUT_FEATURES), jnp.float32),
        scratch_shapes=[pltpu.VMEM((TN, D_MODEL), jnp.bfloat16)],
        compiler_params=pltpu.CompilerParams(
            dimension_semantics=("arbitrary",),
            vmem_limit_bytes=63 * 1024 * 1024,
        ),
    )(xbf, W_base, b2, B_all, hw)
    return out.reshape(lead + (OUT_FEATURES,))
